# Initial kernel scaffold; baseline (speedup 1.0000x reference)
import jax
import jax.numpy as jnp
from jax.experimental import pallas as pl


def kernel(adj_mashup, adj_api, mashup_text_emb, api_text_emb, mashup_emb_w, api_emb_w, text_W, text_b):
    raise NotImplementedError("write your pallas kernel here")


# speedup vs baseline: 1.2239x; 1.2239x over previous
import jax
import jax.numpy as jnp
from jax.experimental import pallas as pl
from jax.experimental.pallas import tpu as pltpu


_DOT_DIMS = (((1,), (0,)), ((), ()))


def _choose_blk(n):
    for b in (400, 200, 100, 50, 16, 8):
        if n % b == 0:
            return b
    return n


def _bf16_dot(a, x):
    return jax.lax.dot_general(a, x, _DOT_DIMS, preferred_element_type=jnp.float32)


def _l2n(x):
    ss = jnp.sum(x * x, axis=-1, keepdims=True)
    return x * jax.lax.rsqrt(jnp.maximum(ss, 1e-24))


def _pass1_kernel(a_ref, x_ref, a8_ref, e1_ref):
    a = a_ref[...]
    a8_ref[...] = a.astype(jnp.float8_e4m3fn)
    e1_ref[...] = _bf16_dot(a.astype(jnp.bfloat16), x_ref[...])


def _pass2_kernel(a8_ref, x_ref, o_ref):
    o_ref[...] = _bf16_dot(a8_ref[...].astype(jnp.bfloat16), x_ref[...])


def _pass3_kernel(a8_ref, x_ref, e0_ref, e1_ref, e2_ref, t_ref, w_ref, b_ref, o_ref):
    e3 = _bf16_dot(a8_ref[...].astype(jnp.bfloat16), x_ref[...])
    g = _l2n(e0_ref[...] + e1_ref[...] + e2_ref[...] + e3)
    t = _bf16_dot(t_ref[...].astype(jnp.bfloat16), w_ref[...].astype(jnp.bfloat16))
    t = _l2n(t + b_ref[...])
    o_ref[...] = 0.5 * (g + t)


def _graph_branch(adj, emb_w, txt, w, b):
    n, e = emb_w.shape
    txt_d = txt.shape[1]
    blk = _choose_blk(n)
    grid = (n // blk,)
    params = pltpu.CompilerParams(dimension_semantics=("parallel",))

    row_spec_f32 = pl.BlockSpec((blk, n), lambda i: (i, 0))
    row_spec_f8 = pl.BlockSpec((blk, n), lambda i: (i, 0))
    x_spec = pl.BlockSpec((n, e), lambda i: (0, 0))
    eblk_spec = pl.BlockSpec((blk, e), lambda i: (i, 0))

    a8, e1 = pl.pallas_call(
        _pass1_kernel,
        grid=grid,
        in_specs=[row_spec_f32, x_spec],
        out_specs=[row_spec_f8, eblk_spec],
        out_shape=[
            jax.ShapeDtypeStruct((n, n), jnp.float8_e4m3fn),
            jax.ShapeDtypeStruct((n, e), jnp.float32),
        ],
        compiler_params=params,
    )(adj, emb_w.astype(jnp.bfloat16))

    e2 = pl.pallas_call(
        _pass2_kernel,
        grid=grid,
        in_specs=[row_spec_f8, x_spec],
        out_specs=eblk_spec,
        out_shape=jax.ShapeDtypeStruct((n, e), jnp.float32),
        compiler_params=params,
    )(a8, e1.astype(jnp.bfloat16))

    out = pl.pallas_call(
        _pass3_kernel,
        grid=grid,
        in_specs=[
            row_spec_f8,
            x_spec,
            eblk_spec,
            eblk_spec,
            eblk_spec,
            pl.BlockSpec((blk, txt_d), lambda i: (i, 0)),
            pl.BlockSpec((txt_d, e), lambda i: (0, 0)),
            pl.BlockSpec((1, e), lambda i: (0, 0)),
        ],
        out_specs=eblk_spec,
        out_shape=jax.ShapeDtypeStruct((n, e), jnp.float32),
        compiler_params=params,
    )(a8, e2.astype(jnp.bfloat16), emb_w, e1, e2, txt, w, b.reshape(1, e))

    return out


def kernel(adj_mashup, adj_api, mashup_text_emb, api_text_emb, mashup_emb_w, api_emb_w, text_W, text_b):
    final_mashup = _graph_branch(adj_mashup, mashup_emb_w, mashup_text_emb, text_W, text_b)
    final_api = _graph_branch(adj_api, api_emb_w, api_text_emb, text_W, text_b)
    return (final_mashup, final_api)


# speedup vs baseline: 1.2552x; 1.0256x over previous
import jax
import jax.numpy as jnp
from jax.experimental import pallas as pl
from jax.experimental.pallas import tpu as pltpu


_DOT_DIMS = (((1,), (0,)), ((), ()))
_F8 = jnp.float8_e4m3fn


def _choose_blk(n):
    for b in (400, 200, 100, 50, 16, 8):
        if n % b == 0:
            return b
    return n


def _dot_f32(a, x):
    return jax.lax.dot_general(a, x, _DOT_DIMS, preferred_element_type=jnp.float32)


def _l2n(x):
    ss = jnp.sum(x * x, axis=-1, keepdims=True)
    return x * jax.lax.rsqrt(jnp.maximum(ss, 1e-24))


def _pass1_kernel(a_ref, x_ref, a8_ref, e1_ref, r_ref):
    a = a_ref[...]
    a8_ref[...] = a.astype(_F8)
    ab = a.astype(jnp.bfloat16)
    e1_ref[...] = _dot_f32(ab, x_ref[...])
    r_ref[...] = _dot_f32(ab, jnp.ones(x_ref.shape, jnp.bfloat16))


def _quant_kernel(x_ref, x8_ref, s_ref, c_ref):
    x = x_ref[...]
    c = jnp.mean(x, axis=0, keepdims=True)
    d = x - c
    m = jnp.maximum(jnp.max(jnp.abs(d)), 1e-30)
    x8_ref[...] = (d * (240.0 / m)).astype(_F8)
    s_ref[...] = jnp.full(s_ref.shape, m * (1.0 / 240.0), jnp.float32)
    c_ref[...] = c


def _pass2_kernel(a8_ref, x8_ref, s_ref, c_ref, r_ref, o_ref):
    o_ref[...] = (_dot_f32(a8_ref[...], x8_ref[...]) * s_ref[...]
                  + r_ref[...] * c_ref[...])


def _pass3_kernel(a8_ref, x8_ref, s_ref, c_ref, r_ref, e0_ref, e1_ref, e2_ref,
                  t_ref, w_ref, b_ref, o_ref):
    e3 = (_dot_f32(a8_ref[...], x8_ref[...]) * s_ref[...]
          + r_ref[...] * c_ref[...])
    g = _l2n(e0_ref[...] + e1_ref[...] + e2_ref[...] + e3)
    t = _dot_f32(t_ref[...].astype(jnp.bfloat16), w_ref[...].astype(jnp.bfloat16))
    t = _l2n(t + b_ref[...])
    o_ref[...] = 0.5 * (g + t)


def _quantize(e):
    n, w = e.shape
    return pl.pallas_call(
        _quant_kernel,
        grid=(1,),
        in_specs=[pl.BlockSpec((n, w), lambda i: (0, 0))],
        out_specs=[
            pl.BlockSpec((n, w), lambda i: (0, 0)),
            pl.BlockSpec((1, w), lambda i: (0, 0)),
            pl.BlockSpec((1, w), lambda i: (0, 0)),
        ],
        out_shape=[
            jax.ShapeDtypeStruct((n, w), _F8),
            jax.ShapeDtypeStruct((1, w), jnp.float32),
            jax.ShapeDtypeStruct((1, w), jnp.float32),
        ],
    )(e)


def _graph_branch(adj, emb_w, txt, w, b):
    n, e = emb_w.shape
    txt_d = txt.shape[1]
    blk = _choose_blk(n)
    grid = (n // blk,)
    params = pltpu.CompilerParams(dimension_semantics=("parallel",))

    row_spec = pl.BlockSpec((blk, n), lambda i: (i, 0))
    x_spec = pl.BlockSpec((n, e), lambda i: (0, 0))
    s_spec = pl.BlockSpec((1, e), lambda i: (0, 0))
    eblk_spec = pl.BlockSpec((blk, e), lambda i: (i, 0))

    a8, e1, rsum = pl.pallas_call(
        _pass1_kernel,
        grid=grid,
        in_specs=[row_spec, x_spec],
        out_specs=[row_spec, eblk_spec, eblk_spec],
        out_shape=[
            jax.ShapeDtypeStruct((n, n), _F8),
            jax.ShapeDtypeStruct((n, e), jnp.float32),
            jax.ShapeDtypeStruct((n, e), jnp.float32),
        ],
        compiler_params=params,
    )(adj, emb_w.astype(jnp.bfloat16))

    x8_1, s1, c1 = _quantize(e1)

    e2 = pl.pallas_call(
        _pass2_kernel,
        grid=grid,
        in_specs=[row_spec, x_spec, s_spec, s_spec, eblk_spec],
        out_specs=eblk_spec,
        out_shape=jax.ShapeDtypeStruct((n, e), jnp.float32),
        compiler_params=params,
    )(a8, x8_1, s1, c1, rsum)

    x8_2, s2, c2 = _quantize(e2)

    out = pl.pallas_call(
        _pass3_kernel,
        grid=grid,
        in_specs=[
            row_spec,
            x_spec,
            s_spec,
            s_spec,
            eblk_spec,
            eblk_spec,
            eblk_spec,
            eblk_spec,
            pl.BlockSpec((blk, txt_d), lambda i: (i, 0)),
            pl.BlockSpec((txt_d, e), lambda i: (0, 0)),
            pl.BlockSpec((1, e), lambda i: (0, 0)),
        ],
        out_specs=eblk_spec,
        out_shape=jax.ShapeDtypeStruct((n, e), jnp.float32),
        compiler_params=params,
    )(a8, x8_2, s2, c2, rsum, emb_w, e1, e2, txt, w, b.reshape(1, e))

    return out


def kernel(adj_mashup, adj_api, mashup_text_emb, api_text_emb, mashup_emb_w, api_emb_w, text_W, text_b):
    final_mashup = _graph_branch(adj_mashup, mashup_emb_w, mashup_text_emb, text_W, text_b)
    final_api = _graph_branch(adj_api, api_emb_w, api_text_emb, text_W, text_b)
    return (final_mashup, final_api)


# speedup vs baseline: 1.2870x; 1.0254x over previous
import jax
import jax.numpy as jnp
from jax.experimental import pallas as pl
from jax.experimental.pallas import tpu as pltpu


_DOT_DIMS = (((1,), (0,)), ((), ()))
_F8 = jnp.float8_e4m3fn


def _choose_blk(n):
    return min(512, n)


def _dot_f32(a, x):
    return jax.lax.dot_general(a, x, _DOT_DIMS, preferred_element_type=jnp.float32)


def _l2n(x):
    ss = jnp.sum(x * x, axis=-1, keepdims=True)
    return x * jax.lax.rsqrt(jnp.maximum(ss, 1e-24))


def _pass1_kernel(a_ref, x_ref, a8_ref, e1_ref, r_ref):
    a = a_ref[...]
    a8_ref[...] = a.astype(_F8)
    ab = a.astype(jnp.bfloat16)
    e1_ref[...] = _dot_f32(ab, x_ref[...])
    r_ref[...] = _dot_f32(ab, jnp.ones(x_ref.shape, jnp.bfloat16))


def _quant_kernel(x_ref, x8_ref, s_ref, c_ref):
    x = x_ref[...]
    c = jnp.mean(x, axis=0, keepdims=True)
    d = x - c
    m = jnp.maximum(jnp.max(jnp.abs(d)), 1e-30)
    x8_ref[...] = (d * (240.0 / m)).astype(_F8)
    s_ref[...] = jnp.full(s_ref.shape, m * (1.0 / 240.0), jnp.float32)
    c_ref[...] = c


def _pass2_kernel(a8_ref, x8_ref, s_ref, c_ref, r_ref, o_ref):
    o_ref[...] = (_dot_f32(a8_ref[...], x8_ref[...]) * s_ref[...]
                  + r_ref[...] * c_ref[...])


def _pass3_kernel(a8_ref, x8_ref, s_ref, c_ref, r_ref, e0_ref, e1_ref, e2_ref,
                  t_ref, w_ref, b_ref, o_ref):
    e3 = (_dot_f32(a8_ref[...], x8_ref[...]) * s_ref[...]
          + r_ref[...] * c_ref[...])
    g = _l2n(e0_ref[...] + e1_ref[...] + e2_ref[...] + e3)
    t = _dot_f32(t_ref[...].astype(jnp.bfloat16), w_ref[...].astype(jnp.bfloat16))
    t = _l2n(t + b_ref[...])
    o_ref[...] = 0.5 * (g + t)


def _quantize(e):
    n, w = e.shape
    return pl.pallas_call(
        _quant_kernel,
        grid=(1,),
        in_specs=[pl.BlockSpec((n, w), lambda i: (0, 0))],
        out_specs=[
            pl.BlockSpec((n, w), lambda i: (0, 0)),
            pl.BlockSpec((1, w), lambda i: (0, 0)),
            pl.BlockSpec((1, w), lambda i: (0, 0)),
        ],
        out_shape=[
            jax.ShapeDtypeStruct((n, w), _F8),
            jax.ShapeDtypeStruct((1, w), jnp.float32),
            jax.ShapeDtypeStruct((1, w), jnp.float32),
        ],
    )(e)


def _graph_branch(adj, emb_w, txt, w, b):
    n, e = emb_w.shape
    txt_d = txt.shape[1]
    blk = _choose_blk(n)
    grid = (pl.cdiv(n, blk),)
    params = pltpu.CompilerParams(dimension_semantics=("parallel",))

    row_spec = pl.BlockSpec((blk, n), lambda i: (i, 0))
    x_spec = pl.BlockSpec((n, e), lambda i: (0, 0))
    s_spec = pl.BlockSpec((1, e), lambda i: (0, 0))
    eblk_spec = pl.BlockSpec((blk, e), lambda i: (i, 0))

    a8, e1, rsum = pl.pallas_call(
        _pass1_kernel,
        grid=grid,
        in_specs=[row_spec, x_spec],
        out_specs=[row_spec, eblk_spec, eblk_spec],
        out_shape=[
            jax.ShapeDtypeStruct((n, n), _F8),
            jax.ShapeDtypeStruct((n, e), jnp.float32),
            jax.ShapeDtypeStruct((n, e), jnp.float32),
        ],
        compiler_params=params,
    )(adj, emb_w.astype(jnp.bfloat16))

    x8_1, s1, c1 = _quantize(e1)

    e2 = pl.pallas_call(
        _pass2_kernel,
        grid=grid,
        in_specs=[row_spec, x_spec, s_spec, s_spec, eblk_spec],
        out_specs=eblk_spec,
        out_shape=jax.ShapeDtypeStruct((n, e), jnp.float32),
        compiler_params=params,
    )(a8, x8_1, s1, c1, rsum)

    x8_2, s2, c2 = _quantize(e2)

    out = pl.pallas_call(
        _pass3_kernel,
        grid=grid,
        in_specs=[
            row_spec,
            x_spec,
            s_spec,
            s_spec,
            eblk_spec,
            eblk_spec,
            eblk_spec,
            eblk_spec,
            pl.BlockSpec((blk, txt_d), lambda i: (i, 0)),
            pl.BlockSpec((txt_d, e), lambda i: (0, 0)),
            pl.BlockSpec((1, e), lambda i: (0, 0)),
        ],
        out_specs=eblk_spec,
        out_shape=jax.ShapeDtypeStruct((n, e), jnp.float32),
        compiler_params=params,
    )(a8, x8_2, s2, c2, rsum, emb_w, e1, e2, txt, w, b.reshape(1, e))

    return out


def kernel(adj_mashup, adj_api, mashup_text_emb, api_text_emb, mashup_emb_w, api_emb_w, text_W, text_b):
    final_mashup = _graph_branch(adj_mashup, mashup_emb_w, mashup_text_emb, text_W, text_b)
    final_api = _graph_branch(adj_api, api_emb_w, api_text_emb, text_W, text_b)
    return (final_mashup, final_api)


# speedup vs baseline: 1.3474x; 1.0469x over previous
import jax
import jax.numpy as jnp
from jax.experimental import pallas as pl
from jax.experimental.pallas import tpu as pltpu


_DOT_DIMS = (((1,), (0,)), ((), ()))
_F8 = jnp.float8_e4m3fn


def _dot_f32(a, x):
    return jax.lax.dot_general(a, x, _DOT_DIMS, preferred_element_type=jnp.float32)


def _l2n(x):
    ss = jnp.sum(x * x, axis=-1, keepdims=True)
    return x * jax.lax.rsqrt(jnp.maximum(ss, 1e-24))


def _quantize_to_scratch(x, x8_ref, sc_ref):
    c = jnp.mean(x, axis=0, keepdims=True)
    d = x - c
    m = jnp.maximum(jnp.max(jnp.abs(d)), 1e-30)
    x8_ref[...] = (d * (240.0 / m)).astype(_F8)
    sc_ref[0:1, :] = c
    sc_ref[1:2, :] = jnp.full((1, x.shape[1]), m * (1.0 / 240.0), jnp.float32)


def _scaled_dot(a8_ref, x8_ref, sc_ref, r_ref):
    return (_dot_f32(a8_ref[...], x8_ref[...]) * sc_ref[1:2, :]
            + r_ref[...] * sc_ref[0:1, :])


def _pass1_kernel(a_ref, x_ref, a8_ref, e1_ref, r_ref):
    a = a_ref[...]
    a8_ref[...] = a.astype(_F8)
    ab = a.astype(jnp.bfloat16)
    e1_ref[...] = _dot_f32(ab, x_ref[...])
    r_ref[...] = _dot_f32(ab, jnp.ones(x_ref.shape, jnp.bfloat16))


def _pass2_kernel(a8_ref, x_ref, r_ref, o_ref, x8_ref, sc_ref):
    @pl.when(pl.program_id(0) == 0)
    def _():
        _quantize_to_scratch(x_ref[...], x8_ref, sc_ref)

    o_ref[...] = _scaled_dot(a8_ref, x8_ref, sc_ref, r_ref)


def _pass3_kernel(a8_ref, x_ref, r_ref, e0_ref, e1_ref, e2_ref, t_ref, w_ref,
                  b_ref, o_ref, x8_ref, sc_ref):
    @pl.when(pl.program_id(0) == 0)
    def _():
        _quantize_to_scratch(x_ref[...], x8_ref, sc_ref)

    e3 = _scaled_dot(a8_ref, x8_ref, sc_ref, r_ref)
    g = _l2n(e0_ref[...] + e1_ref[...] + e2_ref[...] + e3)
    t = _dot_f32(t_ref[...].astype(jnp.bfloat16), w_ref[...].astype(jnp.bfloat16))
    t = _l2n(t + b_ref[...])
    o_ref[...] = 0.5 * (g + t)


def _graph_branch(adj, emb_w, txt, w, b):
    n, e = emb_w.shape
    txt_d = txt.shape[1]
    blk1 = min(512, n)
    blk2 = min(1024, n)

    a8, e1, rsum = pl.pallas_call(
        _pass1_kernel,
        grid=(pl.cdiv(n, blk1),),
        in_specs=[
            pl.BlockSpec((blk1, n), lambda i: (i, 0)),
            pl.BlockSpec((n, e), lambda i: (0, 0)),
        ],
        out_specs=[
            pl.BlockSpec((blk1, n), lambda i: (i, 0)),
            pl.BlockSpec((blk1, e), lambda i: (i, 0)),
            pl.BlockSpec((blk1, e), lambda i: (i, 0)),
        ],
        out_shape=[
            jax.ShapeDtypeStruct((n, n), _F8),
            jax.ShapeDtypeStruct((n, e), jnp.float32),
            jax.ShapeDtypeStruct((n, e), jnp.float32),
        ],
        compiler_params=pltpu.CompilerParams(dimension_semantics=("parallel",)),
    )(adj, emb_w.astype(jnp.bfloat16))

    scratch = [
        pltpu.VMEM((n, e), _F8),
        pltpu.VMEM((2, e), jnp.float32),
    ]

    e2 = pl.pallas_call(
        _pass2_kernel,
        grid=(pl.cdiv(n, blk2),),
        in_specs=[
            pl.BlockSpec((blk2, n), lambda i: (i, 0)),
            pl.BlockSpec((n, e), lambda i: (0, 0)),
            pl.BlockSpec((blk2, e), lambda i: (i, 0)),
        ],
        out_specs=pl.BlockSpec((blk2, e), lambda i: (i, 0)),
        out_shape=jax.ShapeDtypeStruct((n, e), jnp.float32),
        scratch_shapes=scratch,
    )(a8, e1, rsum)

    out = pl.pallas_call(
        _pass3_kernel,
        grid=(pl.cdiv(n, blk2),),
        in_specs=[
            pl.BlockSpec((blk2, n), lambda i: (i, 0)),
            pl.BlockSpec((n, e), lambda i: (0, 0)),
            pl.BlockSpec((blk2, e), lambda i: (i, 0)),
            pl.BlockSpec((blk2, e), lambda i: (i, 0)),
            pl.BlockSpec((blk2, e), lambda i: (i, 0)),
            pl.BlockSpec((blk2, e), lambda i: (i, 0)),
            pl.BlockSpec((blk2, txt_d), lambda i: (i, 0)),
            pl.BlockSpec((txt_d, e), lambda i: (0, 0)),
            pl.BlockSpec((1, e), lambda i: (0, 0)),
        ],
        out_specs=pl.BlockSpec((blk2, e), lambda i: (i, 0)),
        out_shape=jax.ShapeDtypeStruct((n, e), jnp.float32),
        scratch_shapes=scratch,
    )(a8, e2, rsum, emb_w, e1, e2, txt, w, b.reshape(1, e))

    return out


def kernel(adj_mashup, adj_api, mashup_text_emb, api_text_emb, mashup_emb_w, api_emb_w, text_W, text_b):
    final_mashup = _graph_branch(adj_mashup, mashup_emb_w, mashup_text_emb, text_W, text_b)
    final_api = _graph_branch(adj_api, api_emb_w, api_text_emb, text_W, text_b)
    return (final_mashup, final_api)


# speedup vs baseline: 1.3604x; 1.0096x over previous
import functools

import jax
import jax.numpy as jnp
from jax.experimental import pallas as pl
from jax.experimental.pallas import tpu as pltpu


_DOT_DIMS = (((1,), (0,)), ((), ()))
_F8 = jnp.float8_e4m3fn


def _dot_f32(a, x):
    return jax.lax.dot_general(a, x, _DOT_DIMS, preferred_element_type=jnp.float32)


def _l2n(x):
    ss = jnp.sum(x * x, axis=-1, keepdims=True)
    return x * jax.lax.rsqrt(jnp.maximum(ss, 1e-24))


def _quantize_to_scratch(x, x8_ref, sc_ref):
    c = jnp.mean(x, axis=0, keepdims=True)
    d = x - c
    m = jnp.maximum(jnp.max(jnp.abs(d)), 1e-30)
    x8_ref[...] = (d * (240.0 / m)).astype(_F8)
    sc_ref[0:1, :] = c
    sc_ref[1:2, :] = jnp.full((1, x.shape[1]), m * (1.0 / 240.0), jnp.float32)


def _scaled_dot(a8_ref, x8_ref, sc_ref, r_ref):
    return (_dot_f32(a8_ref[...], x8_ref[...]) * sc_ref[1:2, :]
            + r_ref[...] * sc_ref[0:1, :])


def _pass1_kernel(a_ref, x_ref, a8_ref, e1_ref, r_ref):
    a = a_ref[...]
    a8_ref[...] = a.astype(_F8)
    ab = a.astype(jnp.bfloat16)
    e1_ref[...] = _dot_f32(ab, x_ref[...])
    r_ref[...] = _dot_f32(ab, jnp.ones(x_ref.shape, jnp.bfloat16))


def _pass23_kernel(a8_ref, e1f_ref, r_ref, e0_ref, e1b_ref, t_ref, w_ref,
                   b_ref, o_ref, e2s_ref, x8_ref, sc_ref, *, n, blk):
    p = pl.program_id(0)
    i = pl.program_id(1)

    @pl.when((p == 0) & (i == 0))
    def _():
        _quantize_to_scratch(e1f_ref[...], x8_ref, sc_ref)

    @pl.when(p == 0)
    def _():
        e2s_ref[pl.ds(i * blk, blk), :] = _scaled_dot(a8_ref, x8_ref, sc_ref,
                                                      r_ref)

    @pl.when((p == 1) & (i == 0))
    def _():
        _quantize_to_scratch(e2s_ref[0:n, :], x8_ref, sc_ref)

    @pl.when(p == 1)
    def _():
        e3 = _scaled_dot(a8_ref, x8_ref, sc_ref, r_ref)
        e2 = e2s_ref[pl.ds(i * blk, blk), :]
        g = _l2n(e0_ref[...] + e1b_ref[...] + e2 + e3)
        t = _dot_f32(t_ref[...].astype(jnp.bfloat16),
                     w_ref[...].astype(jnp.bfloat16))
        t = _l2n(t + b_ref[...])
        o_ref[...] = 0.5 * (g + t)


def _graph_branch(adj, emb_w, txt, w, b):
    n, e = emb_w.shape
    txt_d = txt.shape[1]
    blk1 = min(512, n)
    blk2 = min(1024, n)

    a8, e1, rsum = pl.pallas_call(
        _pass1_kernel,
        grid=(pl.cdiv(n, blk1),),
        in_specs=[
            pl.BlockSpec((blk1, n), lambda i: (i, 0)),
            pl.BlockSpec((n, e), lambda i: (0, 0)),
        ],
        out_specs=[
            pl.BlockSpec((blk1, n), lambda i: (i, 0)),
            pl.BlockSpec((blk1, e), lambda i: (i, 0)),
            pl.BlockSpec((blk1, e), lambda i: (i, 0)),
        ],
        out_shape=[
            jax.ShapeDtypeStruct((n, n), _F8),
            jax.ShapeDtypeStruct((n, e), jnp.float32),
            jax.ShapeDtypeStruct((n, e), jnp.float32),
        ],
        compiler_params=pltpu.CompilerParams(dimension_semantics=("parallel",)),
    )(adj, emb_w.astype(jnp.bfloat16))

    nb = pl.cdiv(n, blk2)
    out = pl.pallas_call(
        functools.partial(_pass23_kernel, n=n, blk=blk2),
        grid=(2, nb),
        in_specs=[
            pl.BlockSpec((blk2, n), lambda p, i: (i, 0)),
            pl.BlockSpec((n, e), lambda p, i: (0, 0)),
            pl.BlockSpec((blk2, e), lambda p, i: (i, 0)),
            pl.BlockSpec((blk2, e), lambda p, i: (i, 0)),
            pl.BlockSpec((blk2, e), lambda p, i: (i, 0)),
            pl.BlockSpec((blk2, txt_d), lambda p, i: (i * p, 0)),
            pl.BlockSpec((txt_d, e), lambda p, i: (0, 0)),
            pl.BlockSpec((1, e), lambda p, i: (0, 0)),
        ],
        out_specs=pl.BlockSpec((blk2, e), lambda p, i: (i * p, 0)),
        out_shape=jax.ShapeDtypeStruct((n, e), jnp.float32),
        scratch_shapes=[
            pltpu.VMEM((nb * blk2, e), jnp.float32),
            pltpu.VMEM((n, e), _F8),
            pltpu.VMEM((2, e), jnp.float32),
        ],
    )(a8, e1, rsum, emb_w, e1, txt, w, b.reshape(1, e))

    return out


def kernel(adj_mashup, adj_api, mashup_text_emb, api_text_emb, mashup_emb_w, api_emb_w, text_W, text_b):
    final_mashup = _graph_branch(adj_mashup, mashup_emb_w, mashup_text_emb, text_W, text_b)
    final_api = _graph_branch(adj_api, api_emb_w, api_text_emb, text_W, text_b)
    return (final_mashup, final_api)


# speedup vs baseline: 1.4190x; 1.0431x over previous
import functools

import jax
import jax.numpy as jnp
from jax.experimental import pallas as pl
from jax.experimental.pallas import tpu as pltpu


_DOT_DIMS = (((1,), (0,)), ((), ()))
_F8 = jnp.float8_e4m3fn


def _dot_f32(a, x):
    return jax.lax.dot_general(a, x, _DOT_DIMS, preferred_element_type=jnp.float32)


def _l2n(x):
    ss = jnp.sum(x * x, axis=-1, keepdims=True)
    return x * jax.lax.rsqrt(jnp.maximum(ss, 1e-24))


def _quantize_to_scratch(x, x8_ref, sc_ref):
    n, e = x.shape
    c = jnp.mean(x, axis=0, keepdims=True)
    d = x - c
    m = jnp.maximum(jnp.max(jnp.abs(d)), 1e-30)
    d8 = (d * (240.0 / m)).astype(_F8)
    ones_col = (jax.lax.broadcasted_iota(jnp.int32, (n, e), 1) == 0).astype(_F8)
    x8_ref[...] = jnp.concatenate([d8, ones_col], axis=1)
    sc_ref[0:1, :] = c
    sc_ref[1:2, :] = jnp.full((1, e), m * (1.0 / 240.0), jnp.float32)


def _scaled_dot(a8_ref, x8_ref, sc_ref):
    e = sc_ref.shape[1]
    y = _dot_f32(a8_ref[...], x8_ref[...])
    return y[:, :e] * sc_ref[1:2, :] + y[:, e:e + 1] * sc_ref[0:1, :]


def _pass1_kernel(a_ref, x_ref, a8_ref, e1_ref):
    a = a_ref[...]
    a8_ref[...] = a.astype(_F8)
    e1_ref[...] = _dot_f32(a.astype(jnp.bfloat16), x_ref[...])


def _pass23_kernel(a8_ref, e1f_ref, e0_ref, e1b_ref, t_ref, w_ref, b_ref,
                   o_ref, e2s_ref, x8_ref, sc_ref, *, n, blk):
    p = pl.program_id(0)
    i = pl.program_id(1)

    @pl.when((p == 0) & (i == 0))
    def _():
        _quantize_to_scratch(e1f_ref[...], x8_ref, sc_ref)

    @pl.when(p == 0)
    def _():
        e2s_ref[pl.ds(i * blk, blk), :] = _scaled_dot(a8_ref, x8_ref, sc_ref)

    @pl.when((p == 1) & (i == 0))
    def _():
        _quantize_to_scratch(e2s_ref[0:n, :], x8_ref, sc_ref)

    @pl.when(p == 1)
    def _():
        e3 = _scaled_dot(a8_ref, x8_ref, sc_ref)
        e2 = e2s_ref[pl.ds(i * blk, blk), :]
        g = _l2n(e0_ref[...] + e1b_ref[...] + e2 + e3)
        t = _dot_f32(t_ref[...].astype(jnp.bfloat16),
                     w_ref[...].astype(jnp.bfloat16))
        t = _l2n(t + b_ref[...])
        o_ref[...] = 0.5 * (g + t)


def _graph_branch(adj, emb_w, txt, w, b):
    n, e = emb_w.shape
    txt_d = txt.shape[1]
    blk1 = min(512, n)
    blk2 = min(1024, n)

    a8, e1 = pl.pallas_call(
        _pass1_kernel,
        grid=(pl.cdiv(n, blk1),),
        in_specs=[
            pl.BlockSpec((blk1, n), lambda i: (i, 0)),
            pl.BlockSpec((n, e), lambda i: (0, 0)),
        ],
        out_specs=[
            pl.BlockSpec((blk1, n), lambda i: (i, 0)),
            pl.BlockSpec((blk1, e), lambda i: (i, 0)),
        ],
        out_shape=[
            jax.ShapeDtypeStruct((n, n), _F8),
            jax.ShapeDtypeStruct((n, e), jnp.float32),
        ],
        compiler_params=pltpu.CompilerParams(dimension_semantics=("parallel",)),
    )(adj, emb_w.astype(jnp.bfloat16))

    nb = pl.cdiv(n, blk2)
    out = pl.pallas_call(
        functools.partial(_pass23_kernel, n=n, blk=blk2),
        grid=(2, nb),
        in_specs=[
            pl.BlockSpec((blk2, n), lambda p, i: (i, 0)),
            pl.BlockSpec((n, e), lambda p, i: (0, 0)),
            pl.BlockSpec((blk2, e), lambda p, i: (i, 0)),
            pl.BlockSpec((blk2, e), lambda p, i: (i, 0)),
            pl.BlockSpec((blk2, txt_d), lambda p, i: (i * p, 0)),
            pl.BlockSpec((txt_d, e), lambda p, i: (0, 0)),
            pl.BlockSpec((1, e), lambda p, i: (0, 0)),
        ],
        out_specs=pl.BlockSpec((blk2, e), lambda p, i: (i * p, 0)),
        out_shape=jax.ShapeDtypeStruct((n, e), jnp.float32),
        scratch_shapes=[
            pltpu.VMEM((nb * blk2, e), jnp.float32),
            pltpu.VMEM((n, 2 * e), _F8),
            pltpu.VMEM((2, e), jnp.float32),
        ],
    )(a8, e1, emb_w, e1, txt, w, b.reshape(1, e))

    return out


def kernel(adj_mashup, adj_api, mashup_text_emb, api_text_emb, mashup_emb_w, api_emb_w, text_W, text_b):
    final_mashup = _graph_branch(adj_mashup, mashup_emb_w, mashup_text_emb, text_W, text_b)
    final_api = _graph_branch(adj_api, api_emb_w, api_text_emb, text_W, text_b)
    return (final_mashup, final_api)


# speedup vs baseline: 1.4365x; 1.0123x over previous
import functools

import jax
import jax.numpy as jnp
from jax.experimental import pallas as pl
from jax.experimental.pallas import tpu as pltpu


_DOT_DIMS = (((1,), (0,)), ((), ()))
_F8 = jnp.float8_e4m3fn


def _dot_f32(a, x):
    return jax.lax.dot_general(a, x, _DOT_DIMS, preferred_element_type=jnp.float32)


def _l2n(x):
    ss = jnp.sum(x * x, axis=-1, keepdims=True)
    return x * jax.lax.rsqrt(jnp.maximum(ss, 1e-24))


def _quantize_to_scratch(x, x8_ref, sc_ref):
    n, e = x.shape
    c = jnp.mean(x, axis=0, keepdims=True)
    d = x - c
    m = jnp.maximum(jnp.max(jnp.abs(d)), 1e-30)
    d8 = (d * (240.0 / m)).astype(_F8)
    ones_col = (jax.lax.broadcasted_iota(jnp.int32, (n, e), 1) == 0).astype(_F8)
    x8_ref[...] = jnp.concatenate([d8, ones_col], axis=1)
    sc_ref[0:1, :] = c
    sc_ref[1:2, :] = jnp.full((1, e), m * (1.0 / 240.0), jnp.float32)


def _scaled_dot(a8_ref, x8_ref, sc_ref):
    e = sc_ref.shape[1]
    y = _dot_f32(a8_ref[...], x8_ref[...])
    return y[:, :e] * sc_ref[1:2, :] + y[:, e:e + 1] * sc_ref[0:1, :]


def _pass1_kernel(a_ref, x_ref, a8_ref, e1_ref):
    a = a_ref[...]
    a8_ref[...] = a.astype(_F8)
    e1_ref[...] = _dot_f32(a.astype(jnp.bfloat16), x_ref[...].astype(jnp.bfloat16))


def _pass23_kernel(a8_ref, e1f_ref, e0_ref, t_ref, w_ref, b_ref,
                   o_ref, e2s_ref, x8_ref, sc_ref, *, n, blk):
    p = pl.program_id(0)
    i = pl.program_id(1)

    @pl.when((p == 0) & (i == 0))
    def _():
        _quantize_to_scratch(e1f_ref[0:n, :], x8_ref, sc_ref)

    @pl.when(p == 0)
    def _():
        e2s_ref[pl.ds(i * blk, blk), :] = _scaled_dot(a8_ref, x8_ref, sc_ref)

    @pl.when((p == 1) & (i == 0))
    def _():
        _quantize_to_scratch(e2s_ref[0:n, :], x8_ref, sc_ref)

    @pl.when(p == 1)
    def _():
        e3 = _scaled_dot(a8_ref, x8_ref, sc_ref)
        e2 = e2s_ref[pl.ds(i * blk, blk), :]
        e1 = e1f_ref[pl.ds(i * blk, blk), :]
        g = _l2n(e0_ref[...] + e1 + e2 + e3)
        t = _dot_f32(t_ref[...].astype(jnp.bfloat16),
                     w_ref[...].astype(jnp.bfloat16))
        t = _l2n(t + b_ref[...])
        o_ref[...] = 0.5 * (g + t)


def _graph_branch(adj, emb_w, txt, w, b):
    n, e = emb_w.shape
    txt_d = txt.shape[1]
    blk1 = min(512, n)
    blk2 = min(1024, n)
    nb = pl.cdiv(n, blk2)
    np2 = nb * blk2

    a8, e1 = pl.pallas_call(
        _pass1_kernel,
        grid=(np2 // blk1,),
        in_specs=[
            pl.BlockSpec((blk1, n), lambda i: (i, 0)),
            pl.BlockSpec((n, e), lambda i: (0, 0)),
        ],
        out_specs=[
            pl.BlockSpec((blk1, n), lambda i: (i, 0)),
            pl.BlockSpec((blk1, e), lambda i: (i, 0)),
        ],
        out_shape=[
            jax.ShapeDtypeStruct((n, n), _F8),
            jax.ShapeDtypeStruct((np2, e), jnp.float32),
        ],
        compiler_params=pltpu.CompilerParams(dimension_semantics=("parallel",)),
    )(adj, emb_w)

    out = pl.pallas_call(
        functools.partial(_pass23_kernel, n=n, blk=blk2),
        grid=(2, nb),
        in_specs=[
            pl.BlockSpec((blk2, n), lambda p, i: (i, 0)),
            pl.BlockSpec((np2, e), lambda p, i: (0, 0)),
            pl.BlockSpec((blk2, e), lambda p, i: (i, 0)),
            pl.BlockSpec((blk2, txt_d), lambda p, i: (i * p, 0)),
            pl.BlockSpec((txt_d, e), lambda p, i: (0, 0)),
            pl.BlockSpec((1, e), lambda p, i: (0, 0)),
        ],
        out_specs=pl.BlockSpec((blk2, e), lambda p, i: (i * p, 0)),
        out_shape=jax.ShapeDtypeStruct((n, e), jnp.float32),
        scratch_shapes=[
            pltpu.VMEM((np2, e), jnp.float32),
            pltpu.VMEM((n, 2 * e), _F8),
            pltpu.VMEM((2, e), jnp.float32),
        ],
    )(a8, e1, emb_w, txt, w, b.reshape(1, e))

    return out


def kernel(adj_mashup, adj_api, mashup_text_emb, api_text_emb, mashup_emb_w, api_emb_w, text_W, text_b):
    final_mashup = _graph_branch(adj_mashup, mashup_emb_w, mashup_text_emb, text_W, text_b)
    final_api = _graph_branch(adj_api, api_emb_w, api_text_emb, text_W, text_b)
    return (final_mashup, final_api)


# speedup vs baseline: 1.4389x; 1.0017x over previous
import functools

import jax
import jax.numpy as jnp
from jax.experimental import pallas as pl
from jax.experimental.pallas import tpu as pltpu


_DOT_DIMS = (((1,), (0,)), ((), ()))
_F8 = jnp.float8_e4m3fn


def _dot_f32(a, x):
    return jax.lax.dot_general(a, x, _DOT_DIMS, preferred_element_type=jnp.float32)


def _l2n(x):
    ss = jnp.sum(x * x, axis=-1, keepdims=True)
    return x * jax.lax.rsqrt(jnp.maximum(ss, 1e-24))


def _quantize_to_scratch(x, x8_ref, sc_ref):
    n, e = x.shape
    c = jnp.mean(x, axis=0, keepdims=True)
    d = x - c
    m = jnp.maximum(jnp.max(jnp.abs(d)), 1e-30)
    d8 = (d * (240.0 / m)).astype(_F8)
    ones_col = (jax.lax.broadcasted_iota(jnp.int32, (n, e), 1) == 0).astype(_F8)
    x8_ref[...] = jnp.concatenate([d8, ones_col], axis=1)
    sc_ref[0:1, :] = c
    sc_ref[1:2, :] = jnp.full((1, e), m * (1.0 / 240.0), jnp.float32)


def _scaled_dot(a8_ref, x8_ref, sc_ref):
    e = sc_ref.shape[1]
    y = _dot_f32(a8_ref[...], x8_ref[...])
    return y[:, :e] * sc_ref[1:2, :] + y[:, e:e + 1] * sc_ref[0:1, :]


def _pass1_kernel(a_ref, x_ref, a8_ref, e1_ref):
    a = a_ref[...]
    a8_ref[...] = a.astype(_F8)
    e1_ref[...] = _dot_f32(a.astype(jnp.bfloat16), x_ref[...].astype(jnp.bfloat16))


def _pass23_body(a8_ref, e1f_ref, e0_ref, t_ref, w_ref, b_ref,
                 o_ref, e2s_ref, x8_ref, sc_ref, *, n, blk):
    p = pl.program_id(0)
    i = pl.program_id(1)

    @pl.when((p == 0) & (i == 0))
    def _():
        _quantize_to_scratch(e1f_ref[0:n, :], x8_ref, sc_ref)

    @pl.when(p == 0)
    def _():
        e2s_ref[pl.ds(i * blk, blk), :] = _scaled_dot(a8_ref, x8_ref, sc_ref)

    @pl.when((p == 1) & (i == 0))
    def _():
        _quantize_to_scratch(e2s_ref[0:n, :], x8_ref, sc_ref)

    @pl.when(p == 1)
    def _():
        e3 = _scaled_dot(a8_ref, x8_ref, sc_ref)
        e2 = e2s_ref[pl.ds(i * blk, blk), :]
        e1 = e1f_ref[pl.ds(i * blk, blk), :]
        g = _l2n(e0_ref[...] + e1 + e2 + e3)
        t = _dot_f32(t_ref[...], w_ref[...])
        t = _l2n(t + b_ref[...])
        o_ref[...] = 0.5 * (g + t)


def _pass23_kernel(a8_ref, e1f_ref, e0_ref, t_ref, w_ref, b_ref,
                   o_ref, e2s_ref, x8_ref, sc_ref, *, n, blk):
    _pass23_body(a8_ref, e1f_ref, e0_ref, t_ref, w_ref, b_ref,
                 o_ref, e2s_ref, x8_ref, sc_ref, n=n, blk=blk)


def _fused_kernel(a2_ref, x2_ref, a18_ref, e1f_ref, e0_ref, t_ref, w_ref,
                  b_ref, o_ref, a28_ref, e12_ref, e2s_ref, x8_ref, sc_ref,
                  *, n, blk):
    a2 = a2_ref[...]
    a28_ref[...] = a2.astype(_F8)
    e12_ref[...] = _dot_f32(a2.astype(jnp.bfloat16),
                            x2_ref[...].astype(jnp.bfloat16))
    _pass23_body(a18_ref, e1f_ref, e0_ref, t_ref, w_ref, b_ref,
                 o_ref, e2s_ref, x8_ref, sc_ref, n=n, blk=blk)


def _sizes(n):
    blk1 = min(512, n)
    blk2 = min(1024, n)
    nb = pl.cdiv(n, blk2)
    np2 = nb * blk2
    return blk1, blk2, nb, np2


def _pass1(adj, emb_w):
    n, e = emb_w.shape
    blk1, _, _, np2 = _sizes(n)
    return pl.pallas_call(
        _pass1_kernel,
        grid=(np2 // blk1,),
        in_specs=[
            pl.BlockSpec((blk1, n), lambda i: (i, 0)),
            pl.BlockSpec((n, e), lambda i: (0, 0)),
        ],
        out_specs=[
            pl.BlockSpec((blk1, n), lambda i: (i, 0)),
            pl.BlockSpec((blk1, e), lambda i: (i, 0)),
        ],
        out_shape=[
            jax.ShapeDtypeStruct((n, n), _F8),
            jax.ShapeDtypeStruct((np2, e), jnp.float32),
        ],
        compiler_params=pltpu.CompilerParams(dimension_semantics=("parallel",)),
    )(adj, emb_w)


def _pass23(a8, e1, emb_w, txt, w, b):
    n, e = emb_w.shape
    txt_d = txt.shape[1]
    _, blk2, nb, np2 = _sizes(n)
    return pl.pallas_call(
        functools.partial(_pass23_kernel, n=n, blk=blk2),
        grid=(2, nb),
        in_specs=[
            pl.BlockSpec((blk2, n), lambda p, i: (i, 0)),
            pl.BlockSpec((np2, e), lambda p, i: (0, 0)),
            pl.BlockSpec((blk2, e), lambda p, i: (i, 0)),
            pl.BlockSpec((blk2, txt_d), lambda p, i: (i * p, 0)),
            pl.BlockSpec((txt_d, e), lambda p, i: (0, 0)),
            pl.BlockSpec((1, e), lambda p, i: (0, 0)),
        ],
        out_specs=pl.BlockSpec((blk2, e), lambda p, i: (i * p, 0)),
        out_shape=jax.ShapeDtypeStruct((n, e), jnp.float32),
        scratch_shapes=[
            pltpu.VMEM((np2, e), jnp.float32),
            pltpu.VMEM((n, 2 * e), _F8),
            pltpu.VMEM((2, e), jnp.float32),
        ],
    )(a8, e1, emb_w, txt, w, b.reshape(1, e))


def _fused(adj_b, emb_b, a8_a, e1_a, emb_a, txt_a, w, b):
    n, e = emb_a.shape
    txt_d = txt_a.shape[1]
    blkf = min(512, n)
    nbf = pl.cdiv(n, blkf)
    blkh = max(8, blkf // 2)
    np2f = nbf * blkf
    _, _, _, np2 = _sizes(n)

    return pl.pallas_call(
        functools.partial(_fused_kernel, n=n, blk=blkf),
        grid=(2, nbf),
        in_specs=[
            pl.BlockSpec((blkh, n), lambda p, i: (p * nbf + i, 0)),
            pl.BlockSpec((n, e), lambda p, i: (0, 0)),
            pl.BlockSpec((blkf, n), lambda p, i: (i, 0)),
            pl.BlockSpec((np2f, e), lambda p, i: (0, 0)),
            pl.BlockSpec((blkf, e), lambda p, i: (i, 0)),
            pl.BlockSpec((blkf, txt_d), lambda p, i: (i * p, 0)),
            pl.BlockSpec((txt_d, e), lambda p, i: (0, 0)),
            pl.BlockSpec((1, e), lambda p, i: (0, 0)),
        ],
        out_specs=[
            pl.BlockSpec((blkf, e), lambda p, i: (i * p, 0)),
            pl.BlockSpec((blkh, n), lambda p, i: (p * nbf + i, 0)),
            pl.BlockSpec((blkh, e), lambda p, i: (p * nbf + i, 0)),
        ],
        out_shape=[
            jax.ShapeDtypeStruct((n, e), jnp.float32),
            jax.ShapeDtypeStruct((n, n), _F8),
            jax.ShapeDtypeStruct((np2, e), jnp.float32),
        ],
        scratch_shapes=[
            pltpu.VMEM((np2f, e), jnp.float32),
            pltpu.VMEM((n, 2 * e), _F8),
            pltpu.VMEM((2, e), jnp.float32),
        ],
    )(adj_b, emb_b, a8_a, e1_a, emb_a, txt_a, w, b.reshape(1, e))


def kernel(adj_mashup, adj_api, mashup_text_emb, api_text_emb, mashup_emb_w, api_emb_w, text_W, text_b):
    txt_m = mashup_text_emb.astype(jnp.bfloat16)
    txt_a = api_text_emb.astype(jnp.bfloat16)
    w_bf = text_W.astype(jnp.bfloat16)
    a8_m, e1_m = _pass1(adj_mashup, mashup_emb_w)
    final_mashup, a8_a, e1_a = _fused(
        adj_api, api_emb_w, a8_m, e1_m, mashup_emb_w, txt_m, w_bf, text_b)
    final_api = _pass23(a8_a, e1_a, api_emb_w, txt_a, w_bf, text_b)
    return (final_mashup, final_api)
